# baseline (speedup 1.0000x reference)
import jax
import jax.numpy as jnp
from jax.experimental import pallas as pl
from jax.experimental.pallas import tpu as pltpu

_BBLK = 8


def kernel(token_ids, matrix):
    b, s = token_ids.shape
    n, d = matrix.shape
    nblocks = b // _BBLK
    indices = token_ids.astype(jnp.int32).reshape(nblocks, _BBLK, s)

    def body(i_ref, x_ref, o_ref):
        for i in range(_BBLK):
            for j in range(s):
                o_ref[i, j] = x_ref[i_ref[0, i, j]]

    return pl.pallas_call(
        body,
        grid=(nblocks,),
        in_specs=[
            pl.BlockSpec(
                (1, _BBLK, s),
                index_map=lambda i: (i, 0, 0),
                memory_space=pltpu.SMEM,
            ),
            pl.BlockSpec((n, d), index_map=lambda i: (0, 0)),
        ],
        out_specs=pl.BlockSpec((_BBLK, s, d), index_map=lambda i: (i, 0, 0)),
        out_shape=jax.ShapeDtypeStruct((b, s, d), matrix.dtype),
    )(indices, matrix)
